# baseline (speedup 1.0000x reference)
import functools

import jax
import jax.numpy as jnp
from jax import lax
from jax.experimental import pallas as pl
from jax.experimental.pallas import tpu as pltpu
from jax.experimental.pallas import tpu_sc as plsc

B = 16384
N_DENSE = 13
F = 26
D = 16
VOCAB = 100000

NC, NS = 2, 16
NW = NC * NS
RT = B // NW
RB = 16
NB = RT // RB
CHUNK_ROWS = 4
CHUNK_IDX = CHUNK_ROWS * F
CHUNKS_PER_BATCH = RB // CHUNK_ROWS
NCHUNK = RT // CHUNK_ROWS


def _body(tab_hbm, fidx_hbm, wf_hbm, c_hbm, out_hbm,
          idx_v, buf0, buf1, wv, cv, out_v, scr, sem0, sem1):
    wid = lax.axis_index("s") * NC + lax.axis_index("c")

    pltpu.sync_copy(fidx_hbm.at[wid], idx_v)
    pltpu.sync_copy(wf_hbm, wv)
    pltpu.sync_copy(c_hbm, cv)

    beta = cv[0, :]
    head = cv[1, :]
    bias1 = cv[2, :]
    hwl = 0.5 * cv[3, :]
    base_vec = beta * head + bias1
    lane = lax.iota(jnp.int32, 16)

    def fire(j, buf, sem):
        for k in range(CHUNKS_PER_BATCH):
            pltpu.async_copy(
                tab_hbm.at[idx_v.at[j * CHUNKS_PER_BATCH + k]],
                buf.at[pl.ds(k * CHUNK_IDX, CHUNK_IDX)],
                sem)

    def drain(buf, sem):
        pltpu.make_async_copy(tab_hbm.at[pl.ds(0, RB * F)], buf, sem).wait()

    def compute(j, buf):
        for r in range(RB):
            s = jnp.zeros((16,), jnp.float32)
            u = jnp.zeros((16,), jnp.float32)
            for f in range(F):
                e = buf[r * F + f, :]
                u = u + e * (wv[f, :] - hwl * e)
                s = s + e
            rv = u + hwl * (s * s) + base_vec
            scr[pl.ds(r * 17, 16)] = rv
        tot = jnp.zeros((16,), jnp.float32)
        lane17 = lane * 17
        for c in range(16):
            tot = tot + plsc.load_gather(scr, [lane17 + c])
        out_v[pl.ds(j * RB, RB)] = 1.0 / (1.0 + jnp.exp(-tot))

    fire(0, buf0, sem0)

    def loop_body(g, carry):
        j0 = 2 * g
        fire(j0 + 1, buf1, sem1)
        drain(buf0, sem0)
        compute(j0, buf0)

        @pl.when(g < NB // 2 - 1)
        def _():
            fire(j0 + 2, buf0, sem0)

        drain(buf1, sem1)
        compute(j0 + 1, buf1)
        return carry

    lax.fori_loop(0, NB // 2, loop_body, 0)
    pltpu.sync_copy(out_v, out_hbm.at[wid])


@functools.partial(
    pl.kernel,
    out_type=jax.ShapeDtypeStruct((NW, RT), jnp.float32),
    mesh=plsc.VectorSubcoreMesh(core_axis_name="c", subcore_axis_name="s"),
    compiler_params=pltpu.CompilerParams(needs_layout_passes=False, use_tc_tiling_on_sc=False),
    scratch_types=[
        pltpu.VMEM((NCHUNK, CHUNK_IDX), jnp.int32),
        pltpu.VMEM((RB * F, D), jnp.float32),
        pltpu.VMEM((RB * F, D), jnp.float32),
        pltpu.VMEM((F, D), jnp.float32),
        pltpu.VMEM((4, 16), jnp.float32),
        pltpu.VMEM((RT,), jnp.float32),
        pltpu.VMEM((RB * 17,), jnp.float32),
        pltpu.SemaphoreType.DMA,
        pltpu.SemaphoreType.DMA,
    ],
)
def _fm_sc(tab_hbm, fidx_hbm, wf_hbm, c_hbm, out_hbm,
           idx_v, buf0, buf1, wv, cv, out_v, scr, sem0, sem1):
    _body(tab_hbm, fidx_hbm, wf_hbm, c_hbm, out_hbm,
          idx_v, buf0, buf1, wv, cv, out_v, scr, sem0, sem1)


def kernel(dense, sparse_idx, tables, ln_gamma, ln_beta, W, b):
    del dense, ln_gamma
    tab2d = tables.reshape(F * VOCAB, D)
    flat_idx = (sparse_idx.astype(jnp.int32)
                + (jnp.arange(F, dtype=jnp.int32) * VOCAB)[None, :])
    fidx = flat_idx.reshape(NW, NCHUNK, CHUNK_IDX)

    w = W[:, 0]
    wf = w[N_DENSE:N_DENSE + F * D].reshape(F, D)
    beta_pad = jnp.zeros((16,), jnp.float32).at[:N_DENSE].set(ln_beta)
    head_pad = jnp.zeros((16,), jnp.float32).at[:N_DENSE].set(w[:N_DENSE])
    bias1 = jnp.zeros((16,), jnp.float32).at[0].set(b[0])
    wcross = jnp.full((16,), w[N_DENSE + F * D], jnp.float32)
    consts = jnp.stack([beta_pad, head_pad, bias1, wcross])

    out = _fm_sc(tab2d, fidx, wf, consts)
    return out.reshape(B, 1)


# speedup vs baseline: 1.0180x; 1.0180x over previous
import functools

import jax
import jax.numpy as jnp
from jax import lax
from jax.experimental import pallas as pl
from jax.experimental.pallas import tpu as pltpu
from jax.experimental.pallas import tpu_sc as plsc

B = 16384
N_DENSE = 13
F = 26
D = 16
VOCAB = 100000

NC, NS = 2, 16
NW = NC * NS
RT = B // NW
BR = 128
NBLK = RT // BR
NG = BR // 16


def _body(tab_hbm, idxt_hbm, wf_hbm, c_hbm, out_hbm,
          idx_v, buf0, buf1, wv, cv, out_v, scr, sem0, sem1):
    wid = lax.axis_index("s") * NC + lax.axis_index("c")

    pltpu.sync_copy(idxt_hbm.at[wid], idx_v)
    pltpu.sync_copy(wf_hbm, wv)
    pltpu.sync_copy(c_hbm, cv)

    beta = cv[0, :]
    head = cv[1, :]
    bias1 = cv[2, :]
    hwl = 0.5 * cv[3, :]
    base_vec = beta * head + bias1
    lane = lax.iota(jnp.int32, 16)

    def fire(j, buf, sem):
        for f in range(F):
            pltpu.async_copy(
                tab_hbm.at[f].at[idx_v.at[f, pl.ds(j * BR, BR)]],
                buf.at[f],
                sem)

    def drain(buf, sem):
        pltpu.make_async_copy(
            tab_hbm.at[pl.ds(0, F), pl.ds(0, BR), :], buf, sem).wait()

    def compute(j, buf):
        def group(g, carry):
            for r in range(16):
                s = jnp.zeros((16,), jnp.float32)
                u = jnp.zeros((16,), jnp.float32)
                for f in range(F):
                    e = buf[f, g * 16 + r, :]
                    u = u + e * (wv[f, :] - hwl * e)
                    s = s + e
                rv = u + hwl * (s * s) + base_vec
                scr[pl.ds(r * 17, 16)] = rv
            tot = jnp.zeros((16,), jnp.float32)
            lane17 = lane * 17
            for c in range(16):
                tot = tot + plsc.load_gather(scr, [lane17 + c])
            out_v[pl.ds(j * BR + g * 16, 16)] = 1.0 / (1.0 + jnp.exp(-tot))
            return carry

        lax.fori_loop(0, NG, group, 0)

    fire(0, buf0, sem0)

    def loop_body(g, carry):
        j0 = 2 * g
        fire(j0 + 1, buf1, sem1)
        drain(buf0, sem0)
        compute(j0, buf0)

        @pl.when(g < NBLK // 2 - 1)
        def _():
            fire(j0 + 2, buf0, sem0)

        drain(buf1, sem1)
        compute(j0 + 1, buf1)
        return carry

    lax.fori_loop(0, NBLK // 2, loop_body, 0)
    pltpu.sync_copy(out_v, out_hbm.at[wid])


@functools.partial(
    pl.kernel,
    out_type=jax.ShapeDtypeStruct((NW, RT), jnp.float32),
    mesh=plsc.VectorSubcoreMesh(core_axis_name="c", subcore_axis_name="s"),
    compiler_params=pltpu.CompilerParams(
        needs_layout_passes=False, use_tc_tiling_on_sc=False),
    scratch_types=[
        pltpu.VMEM((F, RT), jnp.int32),
        pltpu.VMEM((F, BR, D), jnp.float32),
        pltpu.VMEM((F, BR, D), jnp.float32),
        pltpu.VMEM((F, D), jnp.float32),
        pltpu.VMEM((4, 16), jnp.float32),
        pltpu.VMEM((RT,), jnp.float32),
        pltpu.VMEM((16 * 17,), jnp.float32),
        pltpu.SemaphoreType.DMA,
        pltpu.SemaphoreType.DMA,
    ],
)
def _fm_sc(tab_hbm, idxt_hbm, wf_hbm, c_hbm, out_hbm,
           idx_v, buf0, buf1, wv, cv, out_v, scr, sem0, sem1):
    _body(tab_hbm, idxt_hbm, wf_hbm, c_hbm, out_hbm,
          idx_v, buf0, buf1, wv, cv, out_v, scr, sem0, sem1)


def kernel(dense, sparse_idx, tables, ln_gamma, ln_beta, W, b):
    del dense, ln_gamma
    idxt = sparse_idx.astype(jnp.int32).reshape(NW, RT, F).transpose(0, 2, 1)

    w = W[:, 0]
    wf = w[N_DENSE:N_DENSE + F * D].reshape(F, D)
    beta_pad = jnp.zeros((16,), jnp.float32).at[:N_DENSE].set(ln_beta)
    head_pad = jnp.zeros((16,), jnp.float32).at[:N_DENSE].set(w[:N_DENSE])
    bias1 = jnp.zeros((16,), jnp.float32).at[0].set(b[0])
    wcross = jnp.full((16,), w[N_DENSE + F * D], jnp.float32)
    consts = jnp.stack([beta_pad, head_pad, bias1, wcross])

    out = _fm_sc(tables, idxt, wf, consts)
    return out.reshape(B, 1)
